# Initial kernel scaffold; baseline (speedup 1.0000x reference)
import jax
import jax.numpy as jnp
from jax.experimental import pallas as pl


def kernel(u_c, u_t, scale):
    raise NotImplementedError("write your pallas kernel here")


# speedup vs baseline: 13.5939x; 13.5939x over previous
import math

import numpy as np
import jax
import jax.numpy as jnp
from jax import lax
from jax.experimental import pallas as pl

BB, NN, UD = 16, 1024, 128
NPAIRS = NN * (NN - 1) // 2
LOG2F = np.float32(math.log(2.0))
UMIN = np.float32(1e-6)
USPAN = np.float32(np.float32(1.0 - 1e-6) - np.float32(1e-6))
INV_SQRT2 = np.float32(2.0 ** 0.5)
TI, TJ = 256, 1024


def _threefry_xor(idx_u32, key_lo):
    ks0 = np.uint32(0)
    ks1 = np.uint32(key_lo)
    ks2 = np.uint32(ks0 ^ ks1 ^ np.uint32(0x1BD11BDA))
    x0 = jnp.full_like(idx_u32, ks0)
    x1 = idx_u32 + ks1
    rot = ((13, 15, 26, 6), (17, 29, 16, 24))
    ks = (ks0, ks1, ks2)
    for i in range(5):
        for r in rot[i % 2]:
            x0 = x0 + x1
            x1 = (x1 << np.uint32(r)) | (x1 >> np.uint32(32 - r))
            x1 = x1 ^ x0
        x0 = x0 + ks[(i + 1) % 3]
        x1 = x1 + ks[(i + 2) % 3] + np.uint32(i + 1)
    return x0 ^ x1


def _u01(bits):
    f = lax.bitcast_convert_type(
        (bits >> np.uint32(9)) | np.uint32(0x3F800000), jnp.float32)
    return jnp.maximum(UMIN, (f - np.float32(1.0)) * USPAN + UMIN)


def _logitexp(logp):
    pos = jnp.maximum(logp, -LOG2F)
    neg = jnp.minimum(logp, -LOG2F)
    neg_val = neg - jnp.log(np.float32(1.0) - jnp.exp(neg))
    y = jnp.exp(-pos)
    ym1 = y - np.float32(1.0)
    em1 = jnp.where(ym1 == np.float32(0.0), -pos, ym1 * (-pos) / jnp.log(y))
    pos_val = -jnp.log(jnp.maximum(em1, np.float32(1e-20)))
    return pos_val + neg_val


def _sample(noise_logits):
    return jax.nn.sigmoid(noise_logits / np.float32(0.3))


def _stats_kernel(lc_ref, mi_ref, rk_out, misc_out):
    b = pl.program_id(0)
    lc_row = lc_ref[0]
    lc_col = lc_row.T
    ii = lax.broadcasted_iota(jnp.int32, (NN, NN), 0)
    jj = lax.broadcasted_iota(jnp.int32, (NN, NN), 1)
    cmp = jnp.logical_or(lc_col < lc_row,
                         jnp.logical_and(lc_col == lc_row, ii < jj))
    cnt = jnp.where(cmp, np.float32(1.0), np.float32(0.0))
    rank_row = jnp.sum(cnt, axis=0, keepdims=True).astype(jnp.int32)
    rk_out[0] = rank_row
    s2 = mi_ref[0, 0:1, 0:2]
    sc = mi_ref[0, 0:1, 2:3]
    lg = _logitexp((s2 * np.float32(-0.5)) / sc)
    idx = (b * 2 + lax.broadcasted_iota(jnp.int32, (1, 2), 1)).astype(jnp.uint32)
    u = _u01(_threefry_xor(idx, 2))
    v = _sample(jnp.log(u) - jnp.log(np.float32(1.0) - u) + lg)
    misc_out[0] = jnp.zeros((1, 128), jnp.float32)
    misc_out[0, 0:1, 0:2] = v
    misc_out[0, 0:1, 8:9] = sc


def _graph_kernel(lci_ref, rki_ref, lcj_ref, rkj_ref, misc_ref, g_out, bip_out):
    b = pl.program_id(0)
    i = pl.program_id(1)
    j = pl.program_id(2)
    lci = lci_ref[0].T
    rki = rki_ref[0].T
    lcj = lcj_ref[0]
    rkj = rkj_ref[0]
    sc = misc_ref[0, 0:1, 8:9]
    d = lci - lcj
    lg = _logitexp((d * d * np.float32(-0.5)) / sc)
    mask = rki < rkj
    rowoff = rki * NN - (rki * (rki + 1)) // 2 - rki - 1
    k = rowoff + rkj
    idx = (b * NPAIRS + k).astype(jnp.uint32)
    u = _u01(_threefry_xor(idx, 1))
    val = _sample(jnp.log(u) - jnp.log(np.float32(1.0) - u) + lg)
    g_out[0] = jnp.where(mask, val, np.float32(0.0))
    first = jnp.logical_and(i == 0, j == 0)

    @pl.when(first)
    def _():
        ri = lax.broadcasted_iota(jnp.int32, (TI, TJ), 0)
        ci = lax.broadcasted_iota(jnp.int32, (TI, TJ), 1)
        v0 = misc_ref[0, 0:1, 0:1]
        v1 = misc_ref[0, 0:1, 1:2]
        t0 = jnp.where((ri == 0) & (ci == 0), v0, np.float32(0.0))
        t0 = jnp.where((ri == 0) & (ci == 1), v1, t0)
        bip_out[0] = t0

    @pl.when(jnp.logical_not(first))
    def _():
        bip_out[0] = jnp.zeros((TI, TJ), jnp.float32)


def kernel(u_c, u_t, scale):
    lc = jnp.sum(jnp.log(0.5 * lax.erf(u_c / (2.0 ** 0.5)) + 0.5), axis=-1)
    lc3 = lc[:, None, :]
    d2 = u_t[:, 0:1, :] - u_c[:, 0:2, :]
    s2 = jnp.sum(d2 * d2, axis=-1)
    mi = jnp.concatenate(
        [s2, jnp.broadcast_to(scale.astype(jnp.float32), (BB, 1)),
         jnp.zeros((BB, 125), jnp.float32)], axis=-1)[:, None, :]
    rk3, misc3 = pl.pallas_call(
        _stats_kernel,
        grid=(BB,),
        in_specs=[
            pl.BlockSpec((1, 1, NN), lambda b: (b, 0, 0)),
            pl.BlockSpec((1, 1, 128), lambda b: (b, 0, 0)),
        ],
        out_specs=[
            pl.BlockSpec((1, 1, NN), lambda b: (b, 0, 0)),
            pl.BlockSpec((1, 1, 128), lambda b: (b, 0, 0)),
        ],
        out_shape=[
            jax.ShapeDtypeStruct((BB, 1, NN), jnp.int32),
            jax.ShapeDtypeStruct((BB, 1, 128), jnp.float32),
        ],
    )(lc3, mi)
    graph, bip = pl.pallas_call(
        _graph_kernel,
        grid=(BB, NN // TI, NN // TJ),
        in_specs=[
            pl.BlockSpec((1, 1, TI), lambda b, i, j: (b, 0, i)),
            pl.BlockSpec((1, 1, TI), lambda b, i, j: (b, 0, i)),
            pl.BlockSpec((1, 1, TJ), lambda b, i, j: (b, 0, j)),
            pl.BlockSpec((1, 1, TJ), lambda b, i, j: (b, 0, j)),
            pl.BlockSpec((1, 1, 128), lambda b, i, j: (b, 0, 0)),
        ],
        out_specs=[
            pl.BlockSpec((1, TI, TJ), lambda b, i, j: (b, i, j)),
            pl.BlockSpec((1, TI, TJ), lambda b, i, j: (b, i, j)),
        ],
        out_shape=[
            jax.ShapeDtypeStruct((BB, NN, NN), jnp.float32),
            jax.ShapeDtypeStruct((BB, NN, NN), jnp.float32),
        ],
    )(lc3, rk3, lc3, rk3, misc3)
    return (graph, bip)


# speedup vs baseline: 22.6922x; 1.6693x over previous
import math

import numpy as np
import jax
import jax.numpy as jnp
from jax import lax
from jax.experimental import pallas as pl

BB, NN, UD = 16, 1024, 128
NPAIRS = NN * (NN - 1) // 2
LOG2F = np.float32(math.log(2.0))
UMIN = np.float32(1e-6)
USPAN = np.float32(np.float32(1.0 - 1e-6) - np.float32(1e-6))
INV_SQRT2 = np.float32(2.0 ** 0.5)
TI, TJ = 256, 1024


def _threefry_xor(idx_u32, key_lo):
    ks0 = np.uint32(0)
    ks1 = np.uint32(key_lo)
    ks2 = np.uint32(ks0 ^ ks1 ^ np.uint32(0x1BD11BDA))
    x0 = jnp.full_like(idx_u32, ks0)
    x1 = idx_u32 + ks1
    rot = ((13, 15, 26, 6), (17, 29, 16, 24))
    ks = (ks0, ks1, ks2)
    for i in range(5):
        for r in rot[i % 2]:
            x0 = x0 + x1
            x1 = (x1 << np.uint32(r)) | (x1 >> np.uint32(32 - r))
            x1 = x1 ^ x0
        x0 = x0 + ks[(i + 1) % 3]
        x1 = x1 + ks[(i + 2) % 3] + np.uint32(i + 1)
    return x0 ^ x1


def _u01(bits):
    f = lax.bitcast_convert_type(
        (bits >> np.uint32(9)) | np.uint32(0x3F800000), jnp.float32)
    return jnp.maximum(UMIN, (f - np.float32(1.0)) * USPAN + UMIN)


def _logitexp(logp):
    pos = jnp.maximum(logp, -LOG2F)
    neg = jnp.minimum(logp, -LOG2F)
    neg_val = neg - jnp.log(np.float32(1.0) - jnp.exp(neg))
    y = jnp.exp(-pos)
    ym1 = y - np.float32(1.0)
    em1 = jnp.where(ym1 == np.float32(0.0), -pos, ym1 * (-pos) / jnp.log(y))
    pos_val = -jnp.log(jnp.maximum(em1, np.float32(1e-20)))
    return pos_val + neg_val


def _sample(noise_logits):
    return jax.nn.sigmoid(noise_logits / np.float32(0.3))


def _stats_kernel(lc_ref, mi_ref, rk_out, misc_out):
    b = pl.program_id(0)
    lc_row = lc_ref[0]
    lc_col = lc_row.T
    ii = lax.broadcasted_iota(jnp.int32, (NN, NN), 0)
    jj = lax.broadcasted_iota(jnp.int32, (NN, NN), 1)
    cmp = jnp.logical_or(lc_col < lc_row,
                         jnp.logical_and(lc_col == lc_row, ii < jj))
    cnt = jnp.where(cmp, np.float32(1.0), np.float32(0.0))
    rank_row = jnp.sum(cnt, axis=0, keepdims=True).astype(jnp.int32)
    rk_out[0] = rank_row
    s2 = mi_ref[0, 0:1, 0:2]
    sc = mi_ref[0, 0:1, 2:3]
    lg = _logitexp((s2 * np.float32(-0.5)) / sc)
    idx = (b * 2 + lax.broadcasted_iota(jnp.int32, (1, 2), 1)).astype(jnp.uint32)
    u = _u01(_threefry_xor(idx, 2))
    v = _sample(jnp.log(u) - jnp.log(np.float32(1.0) - u) + lg)
    misc_out[0] = jnp.zeros((1, 128), jnp.float32)
    misc_out[0, 0:1, 0:2] = v
    misc_out[0, 0:1, 8:9] = sc


CH_R, CH_C = 8, 512


def _graph_kernel(lci_ref, rki_ref, lcj_ref, rkj_ref, misc_ref, g_out, bip_out):
    b = pl.program_id(0)
    i = pl.program_id(1)
    j = pl.program_id(2)
    lci = lci_ref[0].T
    rki = rki_ref[0].T
    lcj = lcj_ref[0]
    rkj = rkj_ref[0]
    sc = misc_ref[0, 0:1, 8:9]
    rowoff = rki * NN - (rki * (rki + 1)) // 2 - rki - 1
    base = b * NPAIRS
    for r in range(0, TI, CH_R):
        for c in range(0, TJ, CH_C):
            lci_c = lci[r:r + CH_R, 0:1]
            ro_c = rowoff[r:r + CH_R, 0:1]
            rki_c = rki[r:r + CH_R, 0:1]
            lcj_c = lcj[0:1, c:c + CH_C]
            rkj_c = rkj[0:1, c:c + CH_C]
            d = lci_c - lcj_c
            lg = _logitexp((d * d * np.float32(-0.5)) / sc)
            mask = rki_c < rkj_c
            idx = (base + ro_c + rkj_c).astype(jnp.uint32)
            u = _u01(_threefry_xor(idx, 1))
            val = _sample(jnp.log(u) - jnp.log(np.float32(1.0) - u) + lg)
            g_out[0, r:r + CH_R, c:c + CH_C] = jnp.where(
                mask, val, np.float32(0.0))
    first = jnp.logical_and(i == 0, j == 0)

    @pl.when(first)
    def _():
        ri = lax.broadcasted_iota(jnp.int32, (TI, TJ), 0)
        ci = lax.broadcasted_iota(jnp.int32, (TI, TJ), 1)
        v0 = misc_ref[0, 0:1, 0:1]
        v1 = misc_ref[0, 0:1, 1:2]
        t0 = jnp.where((ri == 0) & (ci == 0), v0, np.float32(0.0))
        t0 = jnp.where((ri == 0) & (ci == 1), v1, t0)
        bip_out[0] = t0

    @pl.when(jnp.logical_not(first))
    def _():
        bip_out[0] = jnp.zeros((TI, TJ), jnp.float32)


def kernel(u_c, u_t, scale):
    lc = jnp.sum(jnp.log(0.5 * lax.erf(u_c / (2.0 ** 0.5)) + 0.5), axis=-1)
    lc3 = lc[:, None, :]
    d2 = u_t[:, 0:1, :] - u_c[:, 0:2, :]
    s2 = jnp.sum(d2 * d2, axis=-1)
    mi = jnp.concatenate(
        [s2, jnp.broadcast_to(scale.astype(jnp.float32), (BB, 1)),
         jnp.zeros((BB, 125), jnp.float32)], axis=-1)[:, None, :]
    rk3, misc3 = pl.pallas_call(
        _stats_kernel,
        grid=(BB,),
        in_specs=[
            pl.BlockSpec((1, 1, NN), lambda b: (b, 0, 0)),
            pl.BlockSpec((1, 1, 128), lambda b: (b, 0, 0)),
        ],
        out_specs=[
            pl.BlockSpec((1, 1, NN), lambda b: (b, 0, 0)),
            pl.BlockSpec((1, 1, 128), lambda b: (b, 0, 0)),
        ],
        out_shape=[
            jax.ShapeDtypeStruct((BB, 1, NN), jnp.int32),
            jax.ShapeDtypeStruct((BB, 1, 128), jnp.float32),
        ],
    )(lc3, mi)
    graph, bip = pl.pallas_call(
        _graph_kernel,
        grid=(BB, NN // TI, NN // TJ),
        in_specs=[
            pl.BlockSpec((1, 1, TI), lambda b, i, j: (b, 0, i)),
            pl.BlockSpec((1, 1, TI), lambda b, i, j: (b, 0, i)),
            pl.BlockSpec((1, 1, TJ), lambda b, i, j: (b, 0, j)),
            pl.BlockSpec((1, 1, TJ), lambda b, i, j: (b, 0, j)),
            pl.BlockSpec((1, 1, 128), lambda b, i, j: (b, 0, 0)),
        ],
        out_specs=[
            pl.BlockSpec((1, TI, TJ), lambda b, i, j: (b, i, j)),
            pl.BlockSpec((1, TI, TJ), lambda b, i, j: (b, i, j)),
        ],
        out_shape=[
            jax.ShapeDtypeStruct((BB, NN, NN), jnp.float32),
            jax.ShapeDtypeStruct((BB, NN, NN), jnp.float32),
        ],
    )(lc3, rk3, lc3, rk3, misc3)
    return (graph, bip)


# speedup vs baseline: 27.4004x; 1.2075x over previous
import math

import numpy as np
import jax
import jax.numpy as jnp
from jax import lax
from jax.experimental import pallas as pl
from jax.experimental.pallas import tpu as pltpu

BB, NN, UD = 16, 1024, 128
NPAIRS = NN * (NN - 1) // 2
LOG2F = np.float32(math.log(2.0))
UMIN = np.float32(1e-6)
USPAN = np.float32(np.float32(1.0 - 1e-6) - np.float32(1e-6))
INV_SQRT2 = np.float32(2.0 ** 0.5)
TI, TJ = 256, 256


def _threefry_xor(x1, key_lo):
    ks1 = int(key_lo) & 0xFFFFFFFF
    ks2 = ks1 ^ 0x1BD11BDA
    rot = ((13, 15, 26, 6), (17, 29, 16, 24))
    inj = ((ks1, ks2 + 1), (ks2, 0 + 2), (0, ks1 + 3), (ks1, ks2 + 4),
           (ks2, 0 + 5))
    x0 = x1
    first = True
    for g in range(5):
        for r in rot[g % 2]:
            if first:
                first = False
            else:
                x0 = x0 + x1
            x1 = (x1 << np.uint32(r)) | (x1 >> np.uint32(32 - r))
            x1 = x1 ^ x0
        a0, a1 = inj[g]
        if a0:
            x0 = x0 + np.uint32(a0 & 0xFFFFFFFF)
        x1 = x1 + np.uint32(a1 & 0xFFFFFFFF)
    return x0 ^ x1


def _u01(bits):
    f = lax.bitcast_convert_type(
        (bits >> np.uint32(9)) | np.uint32(0x3F800000), jnp.float32)
    return jnp.maximum(UMIN, (f - np.float32(1.0)) * USPAN + UMIN)


def _logitexp(logp):
    pos = jnp.maximum(logp, -LOG2F)
    neg = jnp.minimum(logp, -LOG2F)
    neg_val = neg - jnp.log(np.float32(1.0) - jnp.exp(neg))
    y = jnp.exp(-pos)
    ym1 = y - np.float32(1.0)
    em1 = jnp.where(ym1 == np.float32(0.0), -pos, ym1 * (-pos) / jnp.log(y))
    pos_val = -jnp.log(jnp.maximum(em1, np.float32(1e-20)))
    return pos_val + neg_val


def _sample(noise_logits):
    return jax.nn.sigmoid(noise_logits / np.float32(0.3))


def _stats_kernel(lc_ref, mi_ref, rk_out, misc_out):
    b = pl.program_id(0)
    lc_row = lc_ref[0]
    lc_col = lc_row.T
    ii = lax.broadcasted_iota(jnp.int32, (NN, NN), 0)
    jj = lax.broadcasted_iota(jnp.int32, (NN, NN), 1)
    cmp = jnp.logical_or(lc_col < lc_row,
                         jnp.logical_and(lc_col == lc_row, ii < jj))
    cnt = jnp.where(cmp, np.float32(1.0), np.float32(0.0))
    rank_row = jnp.sum(cnt, axis=0, keepdims=True).astype(jnp.int32)
    rk_out[0] = rank_row
    s2 = mi_ref[0, 0:1, 0:2]
    sc = mi_ref[0, 0:1, 2:3]
    lg = _logitexp((s2 * np.float32(-0.5)) / sc)
    idx = (b * 2 + 2 + lax.broadcasted_iota(jnp.int32, (1, 2), 1)).astype(jnp.uint32)
    u = _u01(_threefry_xor(idx, 2))
    v = _sample(jnp.log(u) - jnp.log(np.float32(1.0) - u) + lg)
    misc_out[0] = jnp.zeros((1, 128), jnp.float32)
    misc_out[0, 0:1, 0:2] = v
    misc_out[0, 0:1, 8:9] = sc


CH_R = 16
NIB = NN // TI
NSLOT = NIB * (NIB - 1) // 2


def _graph_kernel(lci_ref, rki_ref, lcj_ref, rkj_ref, misc_ref, g_out, bip_out,
                  vbuf):
    b = pl.program_id(0)
    i = pl.program_id(1)
    lci = lci_ref[0].T
    rki = rki_ref[0].T
    lcj = lcj_ref[0]
    rkj = rkj_ref[0]
    sc = misc_ref[0, 0:1, 8:9]
    base1 = b * NPAIRS + 1
    for jj in range(NIB):
        cols = slice(jj * TI, (jj + 1) * TI)
        lcj_s = lcj[0:1, cols]
        rkj_s = rkj[0:1, cols]
        slot_w = jnp.where(i < jj, i * NIB - (i * (i + 1)) // 2 + (jj - i - 1),
                           NSLOT)

        @pl.when(i <= jj)
        def _(lcj_s=lcj_s, rkj_s=rkj_s, slot_w=slot_w, cols=cols):
            for r in range(0, TI, CH_R):
                lci_c = lci[r:r + CH_R, 0:1]
                rki_c = rki[r:r + CH_R, 0:1]
                d = lci_c - lcj_s
                lg = _logitexp((d * d * np.float32(-0.5)) / sc)
                r0 = jnp.minimum(rki_c, rkj_s)
                r1 = jnp.maximum(rki_c, rkj_s)
                k = r0 * NN - (r0 * (r0 + 1)) // 2 - r0 - 1 + r1
                x1 = (base1 + k).astype(jnp.uint32)
                u = _u01(_threefry_xor(x1, 1))
                val = _sample(jnp.log(u) - jnp.log(np.float32(1.0) - u) + lg)
                g_out[0, r:r + CH_R, cols] = jnp.where(
                    rki_c < rkj_s, val, np.float32(0.0))
                vbuf[slot_w, r:r + CH_R, :] = val

        @pl.when(i > jj)
        def _(lcj_s=lcj_s, rkj_s=rkj_s, cols=cols, jj=jj):
            s = jj * NIB - (jj * (jj + 1)) // 2 + (i - jj - 1)
            vt = vbuf[s].T
            g_out[0, :, cols] = jnp.where(rki < rkj_s, vt, np.float32(0.0))

    bip_out[0] = jnp.zeros((TI, NN), jnp.float32)

    @pl.when(i == 0)
    def _():
        ri = lax.broadcasted_iota(jnp.int32, (8, 128), 0)
        ci = lax.broadcasted_iota(jnp.int32, (8, 128), 1)
        v0 = misc_ref[0, 0:1, 0:1]
        v1 = misc_ref[0, 0:1, 1:2]
        t0 = jnp.where((ri == 0) & (ci == 0), v0, np.float32(0.0))
        t0 = jnp.where((ri == 0) & (ci == 1), v1, t0)
        bip_out[0, 0:8, 0:128] = t0


def kernel(u_c, u_t, scale):
    lc = jnp.sum(jnp.log(0.5 * lax.erf(u_c / (2.0 ** 0.5)) + 0.5), axis=-1)
    lc3 = lc[:, None, :]
    d2 = u_t[:, 0:1, :] - u_c[:, 0:2, :]
    s2 = jnp.sum(d2 * d2, axis=-1)
    mi = jnp.concatenate(
        [s2, jnp.broadcast_to(scale.astype(jnp.float32), (BB, 1)),
         jnp.zeros((BB, 125), jnp.float32)], axis=-1)[:, None, :]
    rk3, misc3 = pl.pallas_call(
        _stats_kernel,
        grid=(BB,),
        in_specs=[
            pl.BlockSpec((1, 1, NN), lambda b: (b, 0, 0)),
            pl.BlockSpec((1, 1, 128), lambda b: (b, 0, 0)),
        ],
        out_specs=[
            pl.BlockSpec((1, 1, NN), lambda b: (b, 0, 0)),
            pl.BlockSpec((1, 1, 128), lambda b: (b, 0, 0)),
        ],
        out_shape=[
            jax.ShapeDtypeStruct((BB, 1, NN), jnp.int32),
            jax.ShapeDtypeStruct((BB, 1, 128), jnp.float32),
        ],
    )(lc3, mi)
    graph, bip = pl.pallas_call(
        _graph_kernel,
        grid=(BB, NN // TI),
        in_specs=[
            pl.BlockSpec((1, 1, TI), lambda b, i: (b, 0, i)),
            pl.BlockSpec((1, 1, TI), lambda b, i: (b, 0, i)),
            pl.BlockSpec((1, 1, NN), lambda b, i: (b, 0, 0)),
            pl.BlockSpec((1, 1, NN), lambda b, i: (b, 0, 0)),
            pl.BlockSpec((1, 1, 128), lambda b, i: (b, 0, 0)),
        ],
        out_specs=[
            pl.BlockSpec((1, TI, NN), lambda b, i: (b, i, 0)),
            pl.BlockSpec((1, TI, NN), lambda b, i: (b, i, 0)),
        ],
        out_shape=[
            jax.ShapeDtypeStruct((BB, NN, NN), jnp.float32),
            jax.ShapeDtypeStruct((BB, NN, NN), jnp.float32),
        ],
        scratch_shapes=[
            pltpu.VMEM((NSLOT + 1, TI, TI), jnp.float32),
        ],
    )(lc3, rk3, lc3, rk3, misc3)
    return (graph, bip)


# speedup vs baseline: 30.2486x; 1.1039x over previous
import math

import numpy as np
import jax
import jax.numpy as jnp
from jax import lax
from jax.experimental import pallas as pl
from jax.experimental.pallas import tpu as pltpu

BB, NN, UD = 16, 1024, 128
NPAIRS = NN * (NN - 1) // 2
LOG2F = np.float32(math.log(2.0))
UMIN = np.float32(1e-6)
USPAN = np.float32(np.float32(1.0 - 1e-6) - np.float32(1e-6))
INV_SQRT2 = np.float32(2.0 ** 0.5)
TI, TJ = 256, 256


def _threefry_xor(x1, key_lo):
    ks1 = int(key_lo) & 0xFFFFFFFF
    ks2 = ks1 ^ 0x1BD11BDA
    rot = ((13, 15, 26, 6), (17, 29, 16, 24))
    inj = ((ks1, ks2 + 1), (ks2, 0 + 2), (0, ks1 + 3), (ks1, ks2 + 4),
           (ks2, 0 + 5))
    x0 = x1
    first = True
    for g in range(5):
        for r in rot[g % 2]:
            if first:
                first = False
            else:
                x0 = x0 + x1
            x1 = (x1 << np.uint32(r)) | (x1 >> np.uint32(32 - r))
            x1 = x1 ^ x0
        a0, a1 = inj[g]
        if a0:
            x0 = x0 + np.uint32(a0 & 0xFFFFFFFF)
        x1 = x1 + np.uint32(a1 & 0xFFFFFFFF)
    return x0 ^ x1


def _u01(bits):
    f = lax.bitcast_convert_type(
        (bits >> np.uint32(9)) | np.uint32(0x3F800000), jnp.float32)
    return jnp.maximum(UMIN, (f - np.float32(1.0)) * USPAN + UMIN)


def _logitexp(logp):
    pos = jnp.maximum(logp, -LOG2F)
    neg = jnp.minimum(logp, -LOG2F)
    neg_val = neg - jnp.log(np.float32(1.0) - jnp.exp(neg))
    y = jnp.exp(-pos)
    ym1 = y - np.float32(1.0)
    em1 = jnp.where(ym1 == np.float32(0.0), -pos, ym1 * (-pos) / jnp.log(y))
    pos_val = -jnp.log(jnp.maximum(em1, np.float32(1e-20)))
    return pos_val + neg_val


def _sample(noise_logits):
    return jax.nn.sigmoid(noise_logits / np.float32(0.3))


def _stats_kernel(lc_ref, mi_ref, rk_out, misc_out):
    b = pl.program_id(0)
    lc_row = lc_ref[0]
    lc_col = lc_row.T
    ii = lax.broadcasted_iota(jnp.int32, (NN, NN), 0)
    jj = lax.broadcasted_iota(jnp.int32, (NN, NN), 1)
    cmp = jnp.logical_or(lc_col < lc_row,
                         jnp.logical_and(lc_col == lc_row, ii < jj))
    cnt = jnp.where(cmp, np.float32(1.0), np.float32(0.0))
    rank_row = jnp.sum(cnt, axis=0, keepdims=True).astype(jnp.int32)
    rk_out[0] = rank_row
    s2 = mi_ref[0, 0:1, 0:2]
    sc = mi_ref[0, 0:1, 2:3]
    lg = _logitexp((s2 * np.float32(-0.5)) / sc)
    idx = (b * 2 + 2 + lax.broadcasted_iota(jnp.int32, (1, 2), 1)).astype(jnp.uint32)
    u = _u01(_threefry_xor(idx, 2))
    v = _sample(jnp.log(u) - jnp.log(np.float32(1.0) - u) + lg)
    misc_out[0] = jnp.zeros((1, 128), jnp.float32)
    misc_out[0, 0:1, 0:2] = v
    misc_out[0, 0:1, 8:9] = sc


CH_R = 16
NIB = NN // TI
NSLOT = NIB * (NIB - 1) // 2


def _graph_kernel(lci_ref, rki_ref, lcj_ref, rkj_ref, misc_ref, g_out, bip_out,
                  vbuf):
    b = pl.program_id(0)
    i = pl.program_id(1)
    lci = lci_ref[0].T
    rki = rki_ref[0].T
    lcj = lcj_ref[0]
    rkj = rkj_ref[0]
    sc = misc_ref[0, 0:1, 8:9]
    base1 = b * NPAIRS + 1
    ro_i = base1 + rki * NN - (rki * (rki + 1)) // 2 - rki - 1
    ro_j = base1 + rkj * NN - (rkj * (rkj + 1)) // 2 - rkj - 1
    for jj in range(NIB):
        cols = slice(jj * TI, (jj + 1) * TI)
        lcj_s = lcj[0:1, cols]
        rkj_s = rkj[0:1, cols]
        roj_s = ro_j[0:1, cols]
        slot_w = jnp.where(i < jj, i * NIB - (i * (i + 1)) // 2 + (jj - i - 1),
                           NSLOT)

        @pl.when(i <= jj)
        def _(lcj_s=lcj_s, rkj_s=rkj_s, roj_s=roj_s, slot_w=slot_w, cols=cols):
            for r in range(0, TI, CH_R):
                lci_c = lci[r:r + CH_R, 0:1]
                rki_c = rki[r:r + CH_R, 0:1]
                roi_c = ro_i[r:r + CH_R, 0:1]
                d = lci_c - lcj_s
                lg = _logitexp((d * d * np.float32(-0.5)) / sc)
                lt = rki_c < rkj_s
                x1 = jnp.where(lt, roi_c + rkj_s, roj_s + rki_c).astype(jnp.uint32)
                u = _u01(_threefry_xor(x1, 1))
                val = _sample(jnp.log(u) - jnp.log(np.float32(1.0) - u) + lg)
                g_out[0, r:r + CH_R, cols] = jnp.where(lt, val, np.float32(0.0))
                vbuf[slot_w, r:r + CH_R, :] = val

        @pl.when(i > jj)
        def _(lcj_s=lcj_s, rkj_s=rkj_s, cols=cols, jj=jj):
            s = jj * NIB - (jj * (jj + 1)) // 2 + (i - jj - 1)
            vt = vbuf[s].T
            g_out[0, :, cols] = jnp.where(rki < rkj_s, vt, np.float32(0.0))

    bip_out[0] = jnp.zeros((TI, NN), jnp.float32)

    @pl.when(i == 0)
    def _():
        ri = lax.broadcasted_iota(jnp.int32, (8, 128), 0)
        ci = lax.broadcasted_iota(jnp.int32, (8, 128), 1)
        v0 = misc_ref[0, 0:1, 0:1]
        v1 = misc_ref[0, 0:1, 1:2]
        t0 = jnp.where((ri == 0) & (ci == 0), v0, np.float32(0.0))
        t0 = jnp.where((ri == 0) & (ci == 1), v1, t0)
        bip_out[0, 0:8, 0:128] = t0


def kernel(u_c, u_t, scale):
    lc = jnp.sum(jnp.log(0.5 * lax.erf(u_c / (2.0 ** 0.5)) + 0.5), axis=-1)
    lc3 = lc[:, None, :]
    d2 = u_t[:, 0:1, :] - u_c[:, 0:2, :]
    s2 = jnp.sum(d2 * d2, axis=-1)
    mi = jnp.concatenate(
        [s2, jnp.broadcast_to(scale.astype(jnp.float32), (BB, 1)),
         jnp.zeros((BB, 125), jnp.float32)], axis=-1)[:, None, :]
    rk3, misc3 = pl.pallas_call(
        _stats_kernel,
        grid=(BB,),
        in_specs=[
            pl.BlockSpec((1, 1, NN), lambda b: (b, 0, 0)),
            pl.BlockSpec((1, 1, 128), lambda b: (b, 0, 0)),
        ],
        out_specs=[
            pl.BlockSpec((1, 1, NN), lambda b: (b, 0, 0)),
            pl.BlockSpec((1, 1, 128), lambda b: (b, 0, 0)),
        ],
        out_shape=[
            jax.ShapeDtypeStruct((BB, 1, NN), jnp.int32),
            jax.ShapeDtypeStruct((BB, 1, 128), jnp.float32),
        ],
    )(lc3, mi)
    graph, bip = pl.pallas_call(
        _graph_kernel,
        grid=(BB, NN // TI),
        in_specs=[
            pl.BlockSpec((1, 1, TI), lambda b, i: (b, 0, i)),
            pl.BlockSpec((1, 1, TI), lambda b, i: (b, 0, i)),
            pl.BlockSpec((1, 1, NN), lambda b, i: (b, 0, 0)),
            pl.BlockSpec((1, 1, NN), lambda b, i: (b, 0, 0)),
            pl.BlockSpec((1, 1, 128), lambda b, i: (b, 0, 0)),
        ],
        out_specs=[
            pl.BlockSpec((1, TI, NN), lambda b, i: (b, i, 0)),
            pl.BlockSpec((1, TI, NN), lambda b, i: (b, i, 0)),
        ],
        out_shape=[
            jax.ShapeDtypeStruct((BB, NN, NN), jnp.float32),
            jax.ShapeDtypeStruct((BB, NN, NN), jnp.float32),
        ],
        scratch_shapes=[
            pltpu.VMEM((NSLOT + 1, TI, TI), jnp.float32),
        ],
    )(lc3, rk3, lc3, rk3, misc3)
    return (graph, bip)


# speedup vs baseline: 36.0181x; 1.1907x over previous
import math

import numpy as np
import jax
import jax.numpy as jnp
from jax import lax
from jax.experimental import pallas as pl

BB, NN, UD = 16, 1024, 128
NPAIRS = NN * (NN - 1) // 2
LOG2F = np.float32(math.log(2.0))
UMIN = np.float32(1e-6)
USPAN = np.float32(np.float32(1.0 - 1e-6) - np.float32(1e-6))
TI = 256
NIB = NN // TI
CH_R = 8


def _threefry_xor(x1, key_lo):
    ks1 = int(key_lo) & 0xFFFFFFFF
    ks2 = ks1 ^ 0x1BD11BDA
    rot = ((13, 15, 26, 6), (17, 29, 16, 24))
    inj = ((ks1, ks2 + 1), (ks2, 0 + 2), (0, ks1 + 3), (ks1, ks2 + 4),
           (ks2, 0 + 5))
    x0 = x1
    first = True
    for g in range(5):
        for r in rot[g % 2]:
            if first:
                first = False
            else:
                x0 = x0 + x1
            x1 = (x1 << np.uint32(r)) | (x1 >> np.uint32(32 - r))
            x1 = x1 ^ x0
        a0, a1 = inj[g]
        if a0:
            x0 = x0 + np.uint32(a0 & 0xFFFFFFFF)
        x1 = x1 + np.uint32(a1 & 0xFFFFFFFF)
    return x0 ^ x1


def _u01(bits):
    f = lax.bitcast_convert_type(
        (bits >> np.uint32(9)) | np.uint32(0x3F800000), jnp.float32)
    return jnp.maximum(UMIN, (f - np.float32(1.0)) * USPAN + UMIN)


def _logitexp(logp):
    pos = jnp.maximum(logp, -LOG2F)
    neg = jnp.minimum(logp, -LOG2F)
    neg_val = neg - jnp.log(np.float32(1.0) - jnp.exp(neg))
    y = jnp.exp(-pos)
    ym1 = y - np.float32(1.0)
    em1 = jnp.where(ym1 == np.float32(0.0), -pos, ym1 * (-pos) / jnp.log(y))
    pos_val = -jnp.log(jnp.maximum(em1, np.float32(1e-20)))
    return pos_val + neg_val


def _sample(noise_logits):
    return jax.nn.sigmoid(noise_logits / np.float32(0.3))


def _slot(i, j):
    return i * NIB - (i * (i + 1)) // 2 + (j - i - 1)


def _graph_kernel(lc_ref, mi_ref, g_out, bip_out):
    b = pl.program_id(0)
    lc_row = lc_ref[0]
    lc_col = lc_row.T
    sc = mi_ref[0, 0:1, 2:3]
    ii = lax.broadcasted_iota(jnp.int32, (NN, NN), 0)
    jj = lax.broadcasted_iota(jnp.int32, (NN, NN), 1)
    cmp = jnp.logical_or(lc_col < lc_row,
                         jnp.logical_and(lc_col == lc_row, ii < jj))
    cnt = jnp.where(cmp, np.float32(1.0), np.float32(0.0))
    rk_row = jnp.sum(cnt, axis=0, keepdims=True).astype(jnp.int32)
    rk_col = rk_row.T
    base1 = b * NPAIRS + 1
    ro_row = base1 + rk_row * NN - (rk_row * (rk_row + 1)) // 2 - rk_row - 1
    ro_col = ro_row.T

    def compute(r0, h, c0, w, keep):
        lci_b = lc_col[r0:r0 + h, 0:1]
        rki_b = rk_col[r0:r0 + h, 0:1]
        roi_b = ro_col[r0:r0 + h, 0:1]
        csl = slice(c0, c0 + w)
        lcj_s = lc_row[0:1, csl]
        rkj_s = rk_row[0:1, csl]
        roj_s = ro_row[0:1, csl]
        parts = []
        for r in range(0, h, CH_R):
            lci_c = lci_b[r:r + CH_R, 0:1]
            rki_c = rki_b[r:r + CH_R, 0:1]
            roi_c = roi_b[r:r + CH_R, 0:1]
            d = lci_c - lcj_s
            lg = _logitexp((d * d * np.float32(-0.5)) / sc)
            lt = rki_c < rkj_s
            x1 = jnp.where(lt, roi_c + rkj_s,
                           roj_s + rki_c).astype(jnp.uint32)
            u = _u01(_threefry_xor(x1, 1))
            val = _sample(jnp.log(u) - jnp.log(np.float32(1.0) - u) + lg)
            g_out[0, r0 + r:r0 + r + CH_R, csl] = jnp.where(
                lt, val, np.float32(0.0))
            if keep:
                parts.append(val)
        return jnp.concatenate(parts, axis=0) if keep else None

    def mirror(v, r0, c0):
        h, w = v.shape[1], v.shape[0]
        rki_b = rk_col[r0:r0 + h, 0:1]
        rkj_s = rk_row[0:1, c0:c0 + w]
        g_out[0, r0:r0 + h, c0:c0 + w] = jnp.where(
            rki_b < rkj_s, v.T, np.float32(0.0))

    HB = TI // 2
    vals = {}
    for i in range(NIB):
        r0 = i * TI
        for i0 in range(i):
            mirror(vals.pop((i0, i)), r0, i0 * TI)
        v = compute(r0, HB, r0 + HB, HB, True)
        mirror(v, r0 + HB, r0)
        compute(r0, HB, r0, HB, False)
        compute(r0 + HB, HB, r0 + HB, HB, False)
        sbs = list(range(i + 1, NIB))
        for g0 in range(0, len(sbs), 2):
            group = sbs[g0:g0 + 2]
            c0 = group[0] * TI
            w = len(group) * TI
            v = compute(r0, TI, c0, w, True)
            for sj in group:
                vals[(i, sj)] = v[:, sj * TI - c0:sj * TI - c0 + TI]

    s2 = mi_ref[0, 0:1, 0:2]
    lg2 = _logitexp((s2 * np.float32(-0.5)) / sc)
    idx = (b * 2 + 2
           + lax.broadcasted_iota(jnp.int32, (1, 2), 1)).astype(jnp.uint32)
    ub = _u01(_threefry_xor(idx, 2))
    v2 = _sample(jnp.log(ub) - jnp.log(np.float32(1.0) - ub) + lg2)
    bip_out[0] = jnp.zeros((NN, NN), jnp.float32)
    ri = lax.broadcasted_iota(jnp.int32, (8, 128), 0)
    ci = lax.broadcasted_iota(jnp.int32, (8, 128), 1)
    t0 = jnp.where((ri == 0) & (ci == 0), v2[0:1, 0:1], np.float32(0.0))
    t0 = jnp.where((ri == 0) & (ci == 1), v2[0:1, 1:2], t0)
    bip_out[0, 0:8, 0:128] = t0


def kernel(u_c, u_t, scale):
    lc = jnp.sum(jnp.log(0.5 * lax.erf(u_c / (2.0 ** 0.5)) + 0.5), axis=-1)
    lc3 = lc[:, None, :]
    d2 = u_t[:, 0:1, :] - u_c[:, 0:2, :]
    s2 = jnp.sum(d2 * d2, axis=-1)
    mi = jnp.concatenate(
        [s2, jnp.broadcast_to(scale.astype(jnp.float32), (BB, 1)),
         jnp.zeros((BB, 125), jnp.float32)], axis=-1)[:, None, :]
    graph, bip = pl.pallas_call(
        _graph_kernel,
        grid=(BB,),
        in_specs=[
            pl.BlockSpec((1, 1, NN), lambda b: (b, 0, 0)),
            pl.BlockSpec((1, 1, 128), lambda b: (b, 0, 0)),
        ],
        out_specs=[
            pl.BlockSpec((1, NN, NN), lambda b: (b, 0, 0)),
            pl.BlockSpec((1, NN, NN), lambda b: (b, 0, 0)),
        ],
        out_shape=[
            jax.ShapeDtypeStruct((BB, NN, NN), jnp.float32),
            jax.ShapeDtypeStruct((BB, NN, NN), jnp.float32),
        ],
    )(lc3, mi)
    return (graph, bip)


# speedup vs baseline: 36.6700x; 1.0181x over previous
import math

import numpy as np
import jax
import jax.numpy as jnp
from jax import lax
from jax.experimental import pallas as pl

BB, NN, UD = 16, 1024, 128
NPAIRS = NN * (NN - 1) // 2
LOG2F = np.float32(math.log(2.0))
UMIN = np.float32(1e-6)
USPAN = np.float32(np.float32(1.0 - 1e-6) - np.float32(1e-6))
TI = 256
NIB = NN // TI
CH_R = 16


def _threefry_xor(x1, key_lo):
    ks1 = int(key_lo) & 0xFFFFFFFF
    ks2 = ks1 ^ 0x1BD11BDA
    rot = ((13, 15, 26, 6), (17, 29, 16, 24))
    inj = ((ks1, ks2 + 1), (ks2, 0 + 2), (0, ks1 + 3), (ks1, ks2 + 4),
           (ks2, 0 + 5))
    x0 = x1
    first = True
    for g in range(5):
        for r in rot[g % 2]:
            if first:
                first = False
            else:
                x0 = x0 + x1
            x1 = (x1 << np.uint32(r)) | (x1 >> np.uint32(32 - r))
            x1 = x1 ^ x0
        a0, a1 = inj[g]
        if a0:
            x0 = x0 + np.uint32(a0 & 0xFFFFFFFF)
        x1 = x1 + np.uint32(a1 & 0xFFFFFFFF)
    return x0 ^ x1


def _u01(bits):
    f = lax.bitcast_convert_type(
        (bits >> np.uint32(9)) | np.uint32(0x3F800000), jnp.float32)
    return jnp.maximum(UMIN, (f - np.float32(1.0)) * USPAN + UMIN)


def _logitexp(logp):
    pos = jnp.maximum(logp, -LOG2F)
    neg = jnp.minimum(logp, -LOG2F)
    neg_val = neg - jnp.log(np.float32(1.0) - jnp.exp(neg))
    y = jnp.exp(-pos)
    ym1 = y - np.float32(1.0)
    em1 = jnp.where(ym1 == np.float32(0.0), -pos, ym1 * (-pos) / jnp.log(y))
    pos_val = -jnp.log(jnp.maximum(em1, np.float32(1e-20)))
    return pos_val + neg_val


def _sample(noise_logits):
    return jax.nn.sigmoid(noise_logits / np.float32(0.3))


def _slot(i, j):
    return i * NIB - (i * (i + 1)) // 2 + (j - i - 1)


def _graph_kernel(lc_ref, mi_ref, g_out, bip_out):
    b = pl.program_id(0)
    lc_row = lc_ref[0]
    lc_col = lc_row.T
    sc = mi_ref[0, 0:1, 2:3]
    ii = lax.broadcasted_iota(jnp.int32, (NN, NN), 0)
    jj = lax.broadcasted_iota(jnp.int32, (NN, NN), 1)
    cmp = jnp.logical_or(lc_col < lc_row,
                         jnp.logical_and(lc_col == lc_row, ii < jj))
    cnt = jnp.where(cmp, np.float32(1.0), np.float32(0.0))
    rk_row = jnp.sum(cnt, axis=0, keepdims=True).astype(jnp.int32)
    rk_col = rk_row.T
    base1 = b * NPAIRS + 1
    ro_row = base1 + rk_row * NN - (rk_row * (rk_row + 1)) // 2 - rk_row - 1
    ro_col = ro_row.T

    def compute(r0, h, c0, w, keep):
        lci_b = lc_col[r0:r0 + h, 0:1]
        rki_b = rk_col[r0:r0 + h, 0:1]
        roi_b = ro_col[r0:r0 + h, 0:1]
        csl = slice(c0, c0 + w)
        lcj_s = lc_row[0:1, csl]
        rkj_s = rk_row[0:1, csl]
        roj_s = ro_row[0:1, csl]
        parts = []
        for r in range(0, h, CH_R):
            lci_c = lci_b[r:r + CH_R, 0:1]
            rki_c = rki_b[r:r + CH_R, 0:1]
            roi_c = roi_b[r:r + CH_R, 0:1]
            d = lci_c - lcj_s
            lg = _logitexp((d * d * np.float32(-0.5)) / sc)
            lt = rki_c < rkj_s
            x1 = jnp.where(lt, roi_c + rkj_s,
                           roj_s + rki_c).astype(jnp.uint32)
            u = _u01(_threefry_xor(x1, 1))
            val = _sample(jnp.log(u) - jnp.log(np.float32(1.0) - u) + lg)
            g_out[0, r0 + r:r0 + r + CH_R, csl] = jnp.where(
                lt, val, np.float32(0.0))
            if keep:
                parts.append(val)
        return jnp.concatenate(parts, axis=0) if keep else None

    def mirror(v, r0, c0):
        h, w = v.shape[1], v.shape[0]
        rki_b = rk_col[r0:r0 + h, 0:1]
        rkj_s = rk_row[0:1, c0:c0 + w]
        g_out[0, r0:r0 + h, c0:c0 + w] = jnp.where(
            rki_b < rkj_s, v.T, np.float32(0.0))

    HB = TI // 2
    vals = {}
    for i in range(NIB):
        r0 = i * TI
        for i0 in range(i):
            mirror(vals.pop((i0, i)), r0, i0 * TI)
        v = compute(r0, HB, r0 + HB, HB, True)
        mirror(v, r0 + HB, r0)
        compute(r0, HB, r0, HB, False)
        compute(r0 + HB, HB, r0 + HB, HB, False)
        sbs = list(range(i + 1, NIB))
        for g0 in range(0, len(sbs), 2):
            group = sbs[g0:g0 + 2]
            c0 = group[0] * TI
            w = len(group) * TI
            v = compute(r0, TI, c0, w, True)
            for sj in group:
                vals[(i, sj)] = v[:, sj * TI - c0:sj * TI - c0 + TI]

    s2 = mi_ref[0, 0:1, 0:2]
    lg2 = _logitexp((s2 * np.float32(-0.5)) / sc)
    idx = (b * 2 + 2
           + lax.broadcasted_iota(jnp.int32, (1, 2), 1)).astype(jnp.uint32)
    ub = _u01(_threefry_xor(idx, 2))
    v2 = _sample(jnp.log(ub) - jnp.log(np.float32(1.0) - ub) + lg2)
    bip_out[0] = jnp.zeros((NN, NN), jnp.float32)
    ri = lax.broadcasted_iota(jnp.int32, (8, 128), 0)
    ci = lax.broadcasted_iota(jnp.int32, (8, 128), 1)
    t0 = jnp.where((ri == 0) & (ci == 0), v2[0:1, 0:1], np.float32(0.0))
    t0 = jnp.where((ri == 0) & (ci == 1), v2[0:1, 1:2], t0)
    bip_out[0, 0:8, 0:128] = t0


def kernel(u_c, u_t, scale):
    lc = jnp.sum(jnp.log(0.5 * lax.erf(u_c / (2.0 ** 0.5)) + 0.5), axis=-1)
    lc3 = lc[:, None, :]
    d2 = u_t[:, 0:1, :] - u_c[:, 0:2, :]
    s2 = jnp.sum(d2 * d2, axis=-1)
    mi = jnp.concatenate(
        [s2, jnp.broadcast_to(scale.astype(jnp.float32), (BB, 1)),
         jnp.zeros((BB, 125), jnp.float32)], axis=-1)[:, None, :]
    graph, bip = pl.pallas_call(
        _graph_kernel,
        grid=(BB,),
        in_specs=[
            pl.BlockSpec((1, 1, NN), lambda b: (b, 0, 0)),
            pl.BlockSpec((1, 1, 128), lambda b: (b, 0, 0)),
        ],
        out_specs=[
            pl.BlockSpec((1, NN, NN), lambda b: (b, 0, 0)),
            pl.BlockSpec((1, NN, NN), lambda b: (b, 0, 0)),
        ],
        out_shape=[
            jax.ShapeDtypeStruct((BB, NN, NN), jnp.float32),
            jax.ShapeDtypeStruct((BB, NN, NN), jnp.float32),
        ],
    )(lc3, mi)
    return (graph, bip)


# speedup vs baseline: 36.6780x; 1.0002x over previous
import math

import numpy as np
import jax
import jax.numpy as jnp
from jax import lax
from jax.experimental import pallas as pl

BB, NN, UD = 16, 1024, 128
NPAIRS = NN * (NN - 1) // 2
LOG2F = np.float32(math.log(2.0))
UMIN = np.float32(1e-6)
USPAN = np.float32(np.float32(1.0 - 1e-6) - np.float32(1e-6))
TI = 256
NIB = NN // TI
CH_R = 16


def _threefry_xor(x1, key_lo):
    ks1 = int(key_lo) & 0xFFFFFFFF
    ks2 = ks1 ^ 0x1BD11BDA
    rot = ((13, 15, 26, 6), (17, 29, 16, 24))
    inj = ((ks1, ks2 + 1), (ks2, 0 + 2), (0, ks1 + 3), (ks1, ks2 + 4),
           (ks2, 0 + 5))
    x0 = x1
    first = True
    for g in range(5):
        for r in rot[g % 2]:
            if first:
                first = False
            else:
                x0 = x0 + x1
            x1 = (x1 << np.uint32(r)) | (x1 >> np.uint32(32 - r))
            x1 = x1 ^ x0
        a0, a1 = inj[g]
        if a0:
            x0 = x0 + np.uint32(a0 & 0xFFFFFFFF)
        x1 = x1 + np.uint32(a1 & 0xFFFFFFFF)
    return x0 ^ x1


def _u01(bits):
    f = lax.bitcast_convert_type(
        (bits >> np.uint32(9)) | np.uint32(0x3F800000), jnp.float32)
    return jnp.maximum(UMIN, (f - np.float32(1.0)) * USPAN + UMIN)


def _logitexp(logp):
    pos = jnp.maximum(logp, -LOG2F)
    neg = jnp.minimum(logp, -LOG2F)
    neg_val = neg - jnp.log(np.float32(1.0) - jnp.exp(neg))
    y = jnp.exp(-pos)
    ym1 = y - np.float32(1.0)
    em1 = jnp.where(ym1 == np.float32(0.0), -pos, ym1 * (-pos) / jnp.log(y))
    pos_val = -jnp.log(jnp.maximum(em1, np.float32(1e-20)))
    return pos_val + neg_val


def _sample(noise_logits):
    return jax.nn.sigmoid(noise_logits / np.float32(0.3))


def _graph_kernel(lc_ref, mi_ref, g_out, bip_out):
    b = pl.program_id(0)
    lc_row = lc_ref[0]
    lc_col = lc_row.T
    sc = mi_ref[0, 0:1, 2:3]
    ii = lax.broadcasted_iota(jnp.int32, (NN, NN), 0)
    jj = lax.broadcasted_iota(jnp.int32, (NN, NN), 1)
    cmp = jnp.logical_or(lc_col < lc_row,
                         jnp.logical_and(lc_col == lc_row, ii < jj))
    cnt = jnp.where(cmp, np.float32(1.0), np.float32(0.0))
    rk_row = jnp.sum(cnt, axis=0, keepdims=True).astype(jnp.int32)
    rk_col = rk_row.T
    base1 = b * NPAIRS + 1
    ro_row = base1 + rk_row * NN - (rk_row * (rk_row + 1)) // 2 - rk_row - 1
    ro_col = ro_row.T

    def compute(r0, h, c0, w, keep):
        lci_b = lc_col[r0:r0 + h, 0:1]
        rki_b = rk_col[r0:r0 + h, 0:1]
        roi_b = ro_col[r0:r0 + h, 0:1]
        csl = slice(c0, c0 + w)
        lcj_s = lc_row[0:1, csl]
        rkj_s = rk_row[0:1, csl]
        roj_s = ro_row[0:1, csl]
        parts = []
        for r in range(0, h, CH_R):
            lci_c = lci_b[r:r + CH_R, 0:1]
            rki_c = rki_b[r:r + CH_R, 0:1]
            roi_c = roi_b[r:r + CH_R, 0:1]
            d = lci_c - lcj_s
            lg = _logitexp((d * d * np.float32(-0.5)) / sc)
            lt = rki_c < rkj_s
            x1 = jnp.where(lt, roi_c + rkj_s,
                           roj_s + rki_c).astype(jnp.uint32)
            u = _u01(_threefry_xor(x1, 1))
            val = _sample(jnp.log(u) - jnp.log(np.float32(1.0) - u) + lg)
            g_out[0, r0 + r:r0 + r + CH_R, csl] = jnp.where(
                lt, val, np.float32(0.0))
            if keep:
                parts.append(val)
        return jnp.concatenate(parts, axis=0) if keep else None

    def mirror(v, r0, c0):
        h, w = v.shape[1], v.shape[0]
        rki_b = rk_col[r0:r0 + h, 0:1]
        rkj_s = rk_row[0:1, c0:c0 + w]
        g_out[0, r0:r0 + h, c0:c0 + w] = jnp.where(
            rki_b < rkj_s, v.T, np.float32(0.0))

    HB = TI // 2
    vals = {}
    for i in range(NIB):
        r0 = i * TI
        for i0 in range(i):
            mirror(vals.pop((i0, i)), r0, i0 * TI)
        v = compute(r0, HB, r0 + HB, HB, True)
        mirror(v, r0 + HB, r0)
        compute(r0, HB, r0, HB, False)
        compute(r0 + HB, HB, r0 + HB, HB, False)
        sbs = list(range(i + 1, NIB))
        for g0 in range(0, len(sbs), 2):
            group = sbs[g0:g0 + 2]
            c0 = group[0] * TI
            w = len(group) * TI
            v = compute(r0, TI, c0, w, True)
            for sj in group:
                vals[(i, sj)] = v[:, sj * TI - c0:sj * TI - c0 + TI]

    s2 = mi_ref[0, 0:1, 0:2]
    lg2 = _logitexp((s2 * np.float32(-0.5)) / sc)
    idx = (b * 2 + 2
           + lax.broadcasted_iota(jnp.int32, (1, 2), 1)).astype(jnp.uint32)
    ub = _u01(_threefry_xor(idx, 2))
    v2 = _sample(jnp.log(ub) - jnp.log(np.float32(1.0) - ub) + lg2)
    bip_out[0] = jnp.zeros((NN, NN), jnp.float32)
    ri = lax.broadcasted_iota(jnp.int32, (8, 128), 0)
    ci = lax.broadcasted_iota(jnp.int32, (8, 128), 1)
    t0 = jnp.where((ri == 0) & (ci == 0), v2[0:1, 0:1], np.float32(0.0))
    t0 = jnp.where((ri == 0) & (ci == 1), v2[0:1, 1:2], t0)
    bip_out[0, 0:8, 0:128] = t0


def kernel(u_c, u_t, scale):
    lc = jnp.sum(jnp.log(0.5 * lax.erf(u_c / (2.0 ** 0.5)) + 0.5), axis=-1)
    lc3 = lc[:, None, :]
    d2 = u_t[:, 0:1, :] - u_c[:, 0:2, :]
    s2 = jnp.sum(d2 * d2, axis=-1)
    mi = jnp.concatenate(
        [s2, jnp.broadcast_to(scale.astype(jnp.float32), (BB, 1)),
         jnp.zeros((BB, 125), jnp.float32)], axis=-1)[:, None, :]
    graph, bip = pl.pallas_call(
        _graph_kernel,
        grid=(BB,),
        in_specs=[
            pl.BlockSpec((1, 1, NN), lambda b: (b, 0, 0)),
            pl.BlockSpec((1, 1, 128), lambda b: (b, 0, 0)),
        ],
        out_specs=[
            pl.BlockSpec((1, NN, NN), lambda b: (b, 0, 0)),
            pl.BlockSpec((1, NN, NN), lambda b: (b, 0, 0)),
        ],
        out_shape=[
            jax.ShapeDtypeStruct((BB, NN, NN), jnp.float32),
            jax.ShapeDtypeStruct((BB, NN, NN), jnp.float32),
        ],
    )(lc3, mi)
    return (graph, bip)
